# baseline (speedup 1.0000x reference)
import functools

import jax
import jax.numpy as jnp
from jax import lax
from jax.experimental import pallas as pl
from jax.experimental.pallas import tpu as pltpu
from jax.experimental.pallas import tpu_sc as plsc

N = 10000
E = 320000
D = 128
OUT_C = 64

NC = 2
NS = 16
NW = NC * NS
K = 128
CH = 80
HCH = CH // 2
E_PAD = NW * CH * K
ACC_ROWS = 10240
R = 1000

_mesh = plsc.VectorSubcoreMesh(core_axis_name="c", subcore_axis_name="s")


def _deg_body(dst_hbm, out_hbm, didx, ones_v, zeros_v, acc):
    c = lax.axis_index("c")
    s = lax.axis_index("s")
    wid = c * NS + s
    pltpu.sync_copy(dst_hbm.at[pl.ds(wid * CH, CH)], didx)
    for k in range(8):
        ones_v[pl.ds(16 * k, 16)] = jnp.ones((16,), jnp.float32)
    for k in range(40):
        zeros_v[pl.ds(16 * k, 16)] = jnp.zeros((16,), jnp.float32)
    pltpu.sync_copy(zeros_v, acc.at[pl.ds(s * 640, 640)])
    plsc.subcore_barrier()

    def body(j, carry):
        pltpu.sync_copy(ones_v, acc.at[didx.at[j]], add=True)
        return carry

    lax.fori_loop(0, CH, body, 0)
    plsc.subcore_barrier()
    pltpu.sync_copy(acc.at[pl.ds(s * 640, 640)], out_hbm.at[c, pl.ds(s * 640, 640)])


_deg_kernel = pl.kernel(
    _deg_body,
    out_type=jax.ShapeDtypeStruct((NC, ACC_ROWS), jnp.float32),
    mesh=_mesh,
    scratch_types=[
        pltpu.VMEM((CH, K), jnp.int32),
        pltpu.VMEM((K,), jnp.float32),
        pltpu.VMEM((640,), jnp.float32),
        pltpu.VMEM_SHARED((ACC_ROWS,), jnp.float32),
    ],
)


def _scatter_body(p_hbm, src_hbm, dst_hbm, out_hbm, sidx, didx, rows0, rows1,
                  zbuf, acc, sem0, sem1):
    c = lax.axis_index("c")
    s = lax.axis_index("s")
    wid = c * NS + s
    for r in range(32):
        for q in range(8):
            zbuf[r, pl.ds(16 * q, 16)] = jnp.zeros((16,), jnp.float32)

    def zfire(k, carry):
        pltpu.async_copy(zbuf, acc.at[pl.ds(s * 640 + k * 32, 32)], sem0)
        return carry

    def zdrain(k, carry):
        pltpu.make_async_copy(zbuf, acc.at[pl.ds(s * 640, 32)], sem0).wait()
        return carry

    lax.fori_loop(0, 20, zfire, 0)
    lax.fori_loop(0, 20, zdrain, 0)
    plsc.subcore_barrier()

    def phase(p, carry):
        base = wid * CH + p * HCH
        pltpu.sync_copy(src_hbm.at[pl.ds(base, HCH)], sidx)
        pltpu.sync_copy(dst_hbm.at[pl.ds(base, HCH)], didx)
        pltpu.async_copy(p_hbm.at[sidx.at[0]], rows0, sem0)
        pltpu.async_copy(p_hbm.at[sidx.at[1]], rows1, sem1)

        def body(i, c2):
            j0 = 2 * i
            j1 = j0 + 1
            pltpu.make_async_copy(p_hbm.at[sidx.at[j0]], rows0, sem0).wait()
            pltpu.sync_copy(rows0, acc.at[didx.at[j0]], add=True)
            pltpu.async_copy(p_hbm.at[sidx.at[jnp.minimum(j0 + 2, HCH - 2)]],
                             rows0, sem0)
            pltpu.make_async_copy(p_hbm.at[sidx.at[j1]], rows1, sem1).wait()
            pltpu.sync_copy(rows1, acc.at[didx.at[j1]], add=True)
            pltpu.async_copy(p_hbm.at[sidx.at[jnp.minimum(j1 + 2, HCH - 1)]],
                             rows1, sem1)
            return c2

        lax.fori_loop(0, HCH // 2, body, 0)
        pltpu.make_async_copy(p_hbm.at[sidx.at[0]], rows0, sem0).wait()
        pltpu.make_async_copy(p_hbm.at[sidx.at[1]], rows1, sem1).wait()
        return carry

    lax.fori_loop(0, 2, phase, 0)
    plsc.subcore_barrier()
    pltpu.sync_copy(acc.at[pl.ds(s * 640, 640)], out_hbm.at[c, pl.ds(s * 640, 640)])


_scatter_kernel = pl.kernel(
    _scatter_body,
    out_type=jax.ShapeDtypeStruct((NC, ACC_ROWS, D), jnp.float32),
    mesh=_mesh,
    scratch_types=[
        pltpu.VMEM((HCH, K), jnp.int32),
        pltpu.VMEM((HCH, K), jnp.int32),
        pltpu.VMEM((K, D), jnp.float32),
        pltpu.VMEM((K, D), jnp.float32),
        pltpu.VMEM((32, D), jnp.float32),
        pltpu.VMEM_SHARED((ACC_ROWS, D), jnp.float32),
        pltpu.SemaphoreType.DMA,
        pltpu.SemaphoreType.DMA,
    ],
)


_DOT = functools.partial(jnp.dot, precision=lax.Precision.HIGHEST,
                         preferred_element_type=jnp.float32)


def _tc1_body(x_ref, degt_ref, w1_ref, b1_ref, wg0_ref, p1_ref, dinv_ref):
    d = degt_ref[:, 0:1] + degt_ref[:, 1:2] + 1.0
    dinv = lax.rsqrt(d)
    dinv_ref[...] = dinv
    h0 = jnp.maximum(_DOT(x_ref[...], w1_ref[...]) + b1_ref[...], 0.0)
    p1_ref[...] = dinv * _DOT(h0, wg0_ref[...])


def _tc_mid_body(sp_ref, p_ref, dinv_ref, b_ref, w_ref, out_ref):
    dinv = dinv_ref[...]
    ssum = sp_ref[0] + sp_ref[1] + p_ref[...]
    t = jnp.maximum(dinv * ssum + b_ref[...], 0.0)
    out_ref[...] = dinv * _DOT(t, w_ref[...])


def _tc_fin_body(sp_ref, p_ref, dinv_ref, bg_ref, w2_ref, b2_ref, out_ref):
    dinv = dinv_ref[...]
    ssum = sp_ref[0] + sp_ref[1] + p_ref[...]
    t = jnp.maximum(dinv * ssum + bg_ref[...], 0.0)
    z = _DOT(t, w2_ref[...]) + b2_ref[...]
    m = jnp.max(z, axis=1, keepdims=True)
    ls = jnp.log(jnp.sum(jnp.exp(z - m), axis=1, keepdims=True)) + m
    out_ref[...] = z - ls


def _full(shape):
    return pl.BlockSpec(shape, lambda i: tuple(0 for _ in shape))


_row_spec = pl.BlockSpec((R, D), lambda i: (i, 0))
_sp_spec = pl.BlockSpec((NC, R, D), lambda i: (0, i, 0))
_dinv_spec = pl.BlockSpec((R, 1), lambda i: (i, 0))

_tc1 = pl.pallas_call(
    _tc1_body,
    grid=(N // R,),
    in_specs=[
        _row_spec,
        pl.BlockSpec((R, NC), lambda i: (i, 0)),
        _full((D, D)),
        _full((1, D)),
        _full((D, D)),
    ],
    out_specs=[_row_spec, _dinv_spec],
    out_shape=[
        jax.ShapeDtypeStruct((N, D), jnp.float32),
        jax.ShapeDtypeStruct((N, 1), jnp.float32),
    ],
)

_tc_mid = pl.pallas_call(
    _tc_mid_body,
    grid=(N // R,),
    in_specs=[_sp_spec, _row_spec, _dinv_spec, _full((1, D)), _full((D, D))],
    out_specs=_row_spec,
    out_shape=jax.ShapeDtypeStruct((N, D), jnp.float32),
)

_tc_fin = pl.pallas_call(
    _tc_fin_body,
    grid=(N // R,),
    in_specs=[_sp_spec, _row_spec, _dinv_spec, _full((1, D)),
              _full((D, OUT_C)), _full((1, OUT_C))],
    out_specs=pl.BlockSpec((R, OUT_C), lambda i: (i, 0)),
    out_shape=jax.ShapeDtypeStruct((N, OUT_C), jnp.float32),
)


def kernel(x, edge_index, W1, b1, Wg0, bg0, Wg1, bg1, Wg2, bg2, W2, b2):
    pad = E_PAD - E
    pad_i = jnp.arange(pad, dtype=jnp.int32)
    src = jnp.concatenate([edge_index[0], pad_i % N])
    dst = jnp.concatenate([edge_index[1], N + pad_i % (ACC_ROWS - N)])
    src2d = src.reshape(NW * CH, K)
    dst2d = dst.reshape(NW * CH, K)

    deg_p = _deg_kernel(dst2d)
    degt = deg_p.T[:N]

    p1, dinv = _tc1(x, degt, W1, b1.reshape(1, D), Wg0)
    s1 = _scatter_kernel(p1, src2d, dst2d)
    p2 = _tc_mid(s1, p1, dinv, bg0.reshape(1, D), Wg1)
    s2 = _scatter_kernel(p2, src2d, dst2d)
    p3 = _tc_mid(s2, p2, dinv, bg1.reshape(1, D), Wg2)
    s3 = _scatter_kernel(p3, src2d, dst2d)
    return _tc_fin(s3, p3, dinv, bg2.reshape(1, D), W2, b2.reshape(1, OUT_C))


# speedup vs baseline: 1.0824x; 1.0824x over previous
import functools

import jax
import jax.numpy as jnp
from jax import lax
from jax.experimental import pallas as pl
from jax.experimental.pallas import tpu as pltpu
from jax.experimental.pallas import tpu_sc as plsc

N = 10000
E = 320000
D = 128
OUT_C = 64

NC = 2
NS = 16
NW = NC * NS
K = 128
CH = 80
HCH = CH // 2
E_PAD = NW * CH * K
ACC_ROWS = 10240
R = 2000

_mesh = plsc.VectorSubcoreMesh(core_axis_name="c", subcore_axis_name="s")


def _deg_body(dst_hbm, out_hbm, didx, ones_v, zeros_v, acc):
    c = lax.axis_index("c")
    s = lax.axis_index("s")
    wid = c * NS + s
    pltpu.sync_copy(dst_hbm.at[pl.ds(wid * CH, CH)], didx)
    for k in range(8):
        ones_v[pl.ds(16 * k, 16)] = jnp.ones((16,), jnp.float32)
    for k in range(40):
        zeros_v[pl.ds(16 * k, 16)] = jnp.zeros((16,), jnp.float32)
    pltpu.sync_copy(zeros_v, acc.at[pl.ds(s * 640, 640)])
    plsc.subcore_barrier()

    def body(j, carry):
        pltpu.sync_copy(ones_v, acc.at[didx.at[j]], add=True)
        return carry

    lax.fori_loop(0, CH, body, 0)
    plsc.subcore_barrier()
    pltpu.sync_copy(acc.at[pl.ds(s * 640, 640)], out_hbm.at[c, pl.ds(s * 640, 640)])


_deg_kernel = pl.kernel(
    _deg_body,
    out_type=jax.ShapeDtypeStruct((NC, ACC_ROWS), jnp.float32),
    mesh=_mesh,
    scratch_types=[
        pltpu.VMEM((CH, K), jnp.int32),
        pltpu.VMEM((K,), jnp.float32),
        pltpu.VMEM((640,), jnp.float32),
        pltpu.VMEM_SHARED((ACC_ROWS,), jnp.float32),
    ],
)


def _scatter_body(p_hbm, src_hbm, dst_hbm, out_hbm, sidx, didx, rows0, rows1,
                  zbuf, acc, sem0, sem1):
    c = lax.axis_index("c")
    s = lax.axis_index("s")
    wid = c * NS + s
    for r in range(32):
        for q in range(8):
            zbuf[r, pl.ds(16 * q, 16)] = jnp.zeros((16,), jnp.float32)

    def zfire(k, carry):
        pltpu.async_copy(zbuf, acc.at[pl.ds(s * 640 + k * 32, 32)], sem0)
        return carry

    def zdrain(k, carry):
        pltpu.make_async_copy(zbuf, acc.at[pl.ds(s * 640, 32)], sem0).wait()
        return carry

    lax.fori_loop(0, 20, zfire, 0)
    lax.fori_loop(0, 20, zdrain, 0)
    plsc.subcore_barrier()

    def phase(p, carry):
        base = wid * CH + p * HCH
        pltpu.sync_copy(src_hbm.at[pl.ds(base, HCH)], sidx)
        pltpu.sync_copy(dst_hbm.at[pl.ds(base, HCH)], didx)
        pltpu.async_copy(p_hbm.at[sidx.at[0]], rows0, sem0)
        pltpu.async_copy(p_hbm.at[sidx.at[1]], rows1, sem1)

        def body(i, c2):
            j0 = 2 * i
            j1 = j0 + 1
            pltpu.make_async_copy(p_hbm.at[sidx.at[j0]], rows0, sem0).wait()
            pltpu.sync_copy(rows0, acc.at[didx.at[j0]], add=True)
            pltpu.async_copy(p_hbm.at[sidx.at[jnp.minimum(j0 + 2, HCH - 2)]],
                             rows0, sem0)
            pltpu.make_async_copy(p_hbm.at[sidx.at[j1]], rows1, sem1).wait()
            pltpu.sync_copy(rows1, acc.at[didx.at[j1]], add=True)
            pltpu.async_copy(p_hbm.at[sidx.at[jnp.minimum(j1 + 2, HCH - 1)]],
                             rows1, sem1)
            return c2

        lax.fori_loop(0, HCH // 2, body, 0)
        pltpu.make_async_copy(p_hbm.at[sidx.at[0]], rows0, sem0).wait()
        pltpu.make_async_copy(p_hbm.at[sidx.at[1]], rows1, sem1).wait()
        return carry

    lax.fori_loop(0, 2, phase, 0)
    plsc.subcore_barrier()
    pltpu.sync_copy(acc.at[pl.ds(s * 640, 640)], out_hbm.at[c, pl.ds(s * 640, 640)])


_scatter_kernel = pl.kernel(
    _scatter_body,
    out_type=jax.ShapeDtypeStruct((NC, ACC_ROWS, D), jnp.float32),
    mesh=_mesh,
    scratch_types=[
        pltpu.VMEM((HCH, K), jnp.int32),
        pltpu.VMEM((HCH, K), jnp.int32),
        pltpu.VMEM((K, D), jnp.float32),
        pltpu.VMEM((K, D), jnp.float32),
        pltpu.VMEM((32, D), jnp.float32),
        pltpu.VMEM_SHARED((ACC_ROWS, D), jnp.float32),
        pltpu.SemaphoreType.DMA,
        pltpu.SemaphoreType.DMA,
    ],
)


_DOT = functools.partial(jnp.dot, precision=lax.Precision.HIGHEST,
                         preferred_element_type=jnp.float32)


def _tc_h0_body(x_ref, w1_ref, b1_ref, h0_ref):
    h0_ref[...] = jnp.maximum(_DOT(x_ref[...], w1_ref[...]) + b1_ref[...], 0.0)


def _tc1_body(h0_ref, degt_ref, wg0_ref, p1_ref, dinv_ref):
    d = degt_ref[:, 0:1] + degt_ref[:, 1:2] + 1.0
    dinv = lax.rsqrt(d)
    dinv_ref[...] = dinv
    p1_ref[...] = dinv * _DOT(h0_ref[...], wg0_ref[...])


def _tc_mid_body(sp_ref, p_ref, dinv_ref, b_ref, w_ref, out_ref):
    dinv = dinv_ref[...]
    ssum = sp_ref[0] + sp_ref[1] + p_ref[...]
    t = jnp.maximum(dinv * ssum + b_ref[...], 0.0)
    out_ref[...] = dinv * _DOT(t, w_ref[...])


def _tc_fin_body(sp_ref, p_ref, dinv_ref, bg_ref, w2_ref, b2_ref, out_ref):
    dinv = dinv_ref[...]
    ssum = sp_ref[0] + sp_ref[1] + p_ref[...]
    t = jnp.maximum(dinv * ssum + bg_ref[...], 0.0)
    z = _DOT(t, w2_ref[...]) + b2_ref[...]
    m = jnp.max(z, axis=1, keepdims=True)
    ls = jnp.log(jnp.sum(jnp.exp(z - m), axis=1, keepdims=True)) + m
    out_ref[...] = z - ls


def _full(shape):
    return pl.BlockSpec(shape, lambda i: tuple(0 for _ in shape))


_row_spec = pl.BlockSpec((R, D), lambda i: (i, 0))
_sp_spec = pl.BlockSpec((NC, R, D), lambda i: (0, i, 0))
_dinv_spec = pl.BlockSpec((R, 1), lambda i: (i, 0))

_tc_h0 = pl.pallas_call(
    _tc_h0_body,
    grid=(N // R,),
    in_specs=[_row_spec, _full((D, D)), _full((1, D))],
    out_specs=_row_spec,
    out_shape=jax.ShapeDtypeStruct((N, D), jnp.float32),
)

_tc1 = pl.pallas_call(
    _tc1_body,
    grid=(N // R,),
    in_specs=[
        _row_spec,
        pl.BlockSpec((R, NC), lambda i: (i, 0)),
        _full((D, D)),
    ],
    out_specs=[_row_spec, _dinv_spec],
    out_shape=[
        jax.ShapeDtypeStruct((N, D), jnp.float32),
        jax.ShapeDtypeStruct((N, 1), jnp.float32),
    ],
)

_tc_mid = pl.pallas_call(
    _tc_mid_body,
    grid=(N // R,),
    in_specs=[_sp_spec, _row_spec, _dinv_spec, _full((1, D)), _full((D, D))],
    out_specs=_row_spec,
    out_shape=jax.ShapeDtypeStruct((N, D), jnp.float32),
)

_tc_fin = pl.pallas_call(
    _tc_fin_body,
    grid=(N // R,),
    in_specs=[_sp_spec, _row_spec, _dinv_spec, _full((1, D)),
              _full((D, OUT_C)), _full((1, OUT_C))],
    out_specs=pl.BlockSpec((R, OUT_C), lambda i: (i, 0)),
    out_shape=jax.ShapeDtypeStruct((N, OUT_C), jnp.float32),
)


def kernel(x, edge_index, W1, b1, Wg0, bg0, Wg1, bg1, Wg2, bg2, W2, b2):
    pad = E_PAD - E
    pad_i = jnp.arange(pad, dtype=jnp.int32)
    src = jnp.concatenate([edge_index[0], pad_i % N])
    dst = jnp.concatenate([edge_index[1], N + pad_i % (ACC_ROWS - N)])
    src2d = src.reshape(NW * CH, K)
    dst2d = dst.reshape(NW * CH, K)

    deg_p = _deg_kernel(dst2d)
    h0 = _tc_h0(x, W1, b1.reshape(1, D))
    degt = deg_p.T[:N]

    p1, dinv = _tc1(h0, degt, Wg0)
    s1 = _scatter_kernel(p1, src2d, dst2d)
    p2 = _tc_mid(s1, p1, dinv, bg0.reshape(1, D), Wg1)
    s2 = _scatter_kernel(p2, src2d, dst2d)
    p3 = _tc_mid(s2, p2, dinv, bg1.reshape(1, D), Wg2)
    s3 = _scatter_kernel(p3, src2d, dst2d)
    return _tc_fin(s3, p3, dinv, bg2.reshape(1, D), W2, b2.reshape(1, OUT_C))
